# baseline (speedup 1.0000x reference)
import functools

import jax
import jax.numpy as jnp
from jax import lax
from jax.experimental import pallas as pl
from jax.experimental.pallas import tpu as pltpu
from jax.experimental.pallas import tpu_sc as plsc


@functools.lru_cache(maxsize=None)
def _build_pos_kernel(d, em, h, w, d_sc):
    info = plsc.get_sparse_core_info()
    NC, NS, L = info.num_cores, info.num_subcores, info.num_lanes
    NW = NC * NS
    f2 = 2 * em
    NG = NW // 2
    YPG = h // NG
    DPW = d_sc // 2
    assert h % NG == 0 and d_sc % 2 == 0 and em % L == 0
    mesh = plsc.VectorSubcoreMesh(core_axis_name="c", subcore_axis_name="s")

    @functools.partial(
        pl.kernel,
        mesh=mesh,
        compiler_params=pltpu.CompilerParams(needs_layout_passes=False),
        out_type=jax.ShapeDtypeStruct((d, h, w, f2), jnp.float32),
        scratch_types=[
            pltpu.VMEM((w, em), jnp.float32),
            pltpu.VMEM((h, em), jnp.float32),
            pltpu.VMEM((YPG, w, f2), jnp.float32),
            pltpu.SemaphoreType.DMA,
        ],
    )
    def pos_kernel(col_hbm, row_hbm, out_hbm, col_v, row_v, stripe_v, sem):
        wid = lax.axis_index("s") * NC + lax.axis_index("c")
        g = wid // 2
        half = wid % 2
        pltpu.sync_copy(col_hbm.at[pl.ds(0, w)], col_v)
        pltpu.sync_copy(row_hbm.at[pl.ds(0, h)], row_v)

        rvs = [[row_v[g * YPG + yy, pl.ds(k * L, L)] for k in range(em // L)]
               for yy in range(YPG)]

        def xbody(x, carry):
            for k in range(em // L):
                v = col_v[x, pl.ds(k * L, L)]
                for yy in range(YPG):
                    stripe_v[yy, x, pl.ds(k * L, L)] = v
            for yy in range(YPG):
                for k in range(em // L):
                    stripe_v[yy, x, pl.ds(em + k * L, L)] = rvs[yy][k]
            return carry

        lax.fori_loop(0, w, xbody, 0)

        y0 = g * YPG
        handles = [
            pltpu.async_copy(
                stripe_v,
                out_hbm.at[half * DPW + dd, pl.ds(y0, YPG)],
                sem)
            for dd in range(DPW)
        ]
        for hd in handles:
            hd.wait()

    return pos_kernel


@functools.lru_cache(maxsize=None)
def _build_tc_kernel(d, em, h, w, d_sc):
    f2 = 2 * em

    def body(buf_ref, col_ref, row_ref, out_hbm, scratch, sem):
        del buf_ref
        col = col_ref[...]
        row = row_ref[...]
        scratch[...] = jnp.concatenate(
            [jnp.broadcast_to(col[None, :, :], (h, w, em)),
             jnp.broadcast_to(row[:, None, :], (h, w, em))], axis=-1)
        handles = [
            pltpu.make_async_copy(scratch, out_hbm.at[dd], sem)
            for dd in range(d_sc, d)
        ]
        for hd in handles:
            hd.start()
        for hd in handles:
            hd.wait()

    return pl.pallas_call(
        body,
        out_shape=jax.ShapeDtypeStruct((d, h, w, f2), jnp.float32),
        in_specs=[
            pl.BlockSpec(memory_space=pl.ANY),
            pl.BlockSpec(memory_space=pltpu.VMEM),
            pl.BlockSpec(memory_space=pltpu.VMEM),
        ],
        out_specs=pl.BlockSpec(memory_space=pl.ANY),
        scratch_shapes=[
            pltpu.VMEM((h, w, f2), jnp.float32),
            pltpu.SemaphoreType.DMA,
        ],
        input_output_aliases={0: 0},
    )


def kernel(scan, row_embed, col_embed, dep_embed):
    d, em, h, w = scan.shape
    d_sc = d - 4
    sc_buf = _build_pos_kernel(d, em, h, w, d_sc)(col_embed, row_embed)
    out = _build_tc_kernel(d, em, h, w, d_sc)(
        sc_buf, col_embed[:w], row_embed[:h])
    return out.transpose(0, 3, 1, 2)


# speedup vs baseline: 1.0154x; 1.0154x over previous
import functools

import jax
import jax.numpy as jnp
from jax import lax
from jax.experimental import pallas as pl
from jax.experimental.pallas import tpu as pltpu
from jax.experimental.pallas import tpu_sc as plsc


@functools.lru_cache(maxsize=None)
def _build_pos_kernel(d, em, h, w):
    info = plsc.get_sparse_core_info()
    NC, NS, L = info.num_cores, info.num_subcores, info.num_lanes
    NW = NC * NS
    f2 = 2 * em
    NG = NW // 2
    YPG = h // NG
    DPW = d // 2
    assert h % NG == 0 and d % 2 == 0 and em % L == 0
    mesh = plsc.VectorSubcoreMesh(core_axis_name="c", subcore_axis_name="s")

    @functools.partial(
        pl.kernel,
        mesh=mesh,
        compiler_params=pltpu.CompilerParams(needs_layout_passes=False),
        out_type=jax.ShapeDtypeStruct((d, h, w, f2), jnp.float32),
        scratch_types=[
            pltpu.VMEM((w, em), jnp.float32),
            pltpu.VMEM((h, em), jnp.float32),
            pltpu.VMEM((YPG, w, f2), jnp.float32),
            pltpu.SemaphoreType.DMA,
        ],
    )
    def pos_kernel(col_hbm, row_hbm, out_hbm, col_v, row_v, stripe_v, sem):
        wid = lax.axis_index("s") * NC + lax.axis_index("c")
        g = wid // 2
        half = wid % 2
        pltpu.sync_copy(col_hbm.at[pl.ds(0, w)], col_v)
        pltpu.sync_copy(row_hbm.at[pl.ds(0, h)], row_v)

        y0 = g * YPG
        d0 = half * DPW

        def build_row(yy):
            rv = [row_v[y0 + yy, pl.ds(k * L, L)] for k in range(em // L)]

            def xbody(x, carry):
                for k in range(em // L):
                    stripe_v[yy, x, pl.ds(k * L, L)] = (
                        col_v[x, pl.ds(k * L, L)])
                for k in range(em // L):
                    stripe_v[yy, x, pl.ds(em + k * L, L)] = rv[k]
                return carry

            lax.fori_loop(0, w, xbody, 0)

        def fire_row(yy):
            return [
                pltpu.async_copy(
                    stripe_v.at[yy], out_hbm.at[d0 + dd, y0 + yy], sem)
                for dd in range(DPW)
            ]

        prev = []
        for yy in range(YPG):
            build_row(yy)
            cur = fire_row(yy)
            for hd in prev:
                hd.wait()
            prev = cur
        for hd in prev:
            hd.wait()

    return pos_kernel


def kernel(scan, row_embed, col_embed, dep_embed):
    d, em, h, w = scan.shape
    out = _build_pos_kernel(d, em, h, w)(col_embed, row_embed)
    return out.transpose(0, 3, 1, 2)


# speedup vs baseline: 1.0232x; 1.0077x over previous
import functools

import jax
import jax.numpy as jnp
from jax import lax
from jax.experimental import pallas as pl
from jax.experimental.pallas import tpu as pltpu
from jax.experimental.pallas import tpu_sc as plsc


@functools.lru_cache(maxsize=None)
def _build_pos_kernel(d, em, h, w):
    info = plsc.get_sparse_core_info()
    NC, NS, L = info.num_cores, info.num_subcores, info.num_lanes
    NW = NC * NS
    f2 = 2 * em
    NG = NW // 2
    YPG = h // NG
    DPW = d // 2
    assert h % NG == 0 and d % 2 == 0 and em % L == 0
    mesh = plsc.VectorSubcoreMesh(core_axis_name="c", subcore_axis_name="s")

    @functools.partial(
        pl.kernel,
        mesh=mesh,
        compiler_params=pltpu.CompilerParams(needs_layout_passes=False),
        out_type=jax.ShapeDtypeStruct((d, h, w, f2), jnp.float32),
        scratch_types=[
            pltpu.VMEM((w, em), jnp.float32),
            pltpu.VMEM((h, em), jnp.float32),
            pltpu.VMEM((YPG, w, f2), jnp.float32),
            pltpu.SemaphoreType.DMA,
        ],
    )
    def pos_kernel(col_hbm, row_hbm, out_hbm, col_v, row_v, stripe_v, sem):
        wid = lax.axis_index("s") * NC + lax.axis_index("c")
        g = wid // 2
        half = wid % 2
        pltpu.sync_copy(col_hbm.at[pl.ds(0, w)], col_v)
        pltpu.sync_copy(row_hbm.at[pl.ds(0, h)], row_v)

        rvs = [[row_v[g * YPG + yy, pl.ds(k * L, L)] for k in range(em // L)]
               for yy in range(YPG)]

        def xbody(x, carry):
            for k in range(em // L):
                v = col_v[x, pl.ds(k * L, L)]
                for yy in range(YPG):
                    stripe_v[yy, x, pl.ds(k * L, L)] = v
            for yy in range(YPG):
                for k in range(em // L):
                    stripe_v[yy, x, pl.ds(em + k * L, L)] = rvs[yy][k]
            return carry

        lax.fori_loop(0, w, xbody, 0)

        y0 = g * YPG
        handles = [
            pltpu.async_copy(
                stripe_v,
                out_hbm.at[half * DPW + dd, pl.ds(y0, YPG)],
                sem)
            for dd in range(DPW)
        ]
        for hd in handles:
            hd.wait()

    return pos_kernel


def kernel(scan, row_embed, col_embed, dep_embed):
    d, em, h, w = scan.shape
    out = _build_pos_kernel(d, em, h, w)(col_embed, row_embed)
    return out.transpose(0, 3, 1, 2)
